# baseline (speedup 1.0000x reference)
import functools

import jax
import jax.numpy as jnp
from jax import lax
from jax.experimental import pallas as pl
from jax.experimental.pallas import tpu as pltpu
from jax.experimental.pallas import tpu_sc as plsc

HEADS = 16
KSEL = 9


def _qkv_kernel(x_ref, w_ref, q_ref, kv_ref):
    d = x_ref.shape[1]
    xb = x_ref[...].astype(jnp.bfloat16)
    acc = jnp.dot(xb, w_ref[...], preferred_element_type=jnp.float32)
    q_ref[...] = acc[:, :d]
    kbits = lax.bitcast_convert_type(acc[:, d : 2 * d], jnp.int32)
    vbits = lax.bitcast_convert_type(acc[:, 2 * d :], jnp.int32)
    kv_ref[...] = lax.shift_right_logical(kbits, 16) | (vbits & jnp.int32(-65536))


def _qkv_call(x2, wqkvT):
    g, d = x2.shape
    rows = 512
    grid = (g // rows,)
    return pl.pallas_call(
        _qkv_kernel,
        grid=grid,
        in_specs=[
            pl.BlockSpec((rows, d), lambda i: (i, 0)),
            pl.BlockSpec((d, 3 * d), lambda i: (0, 0)),
        ],
        out_specs=[
            pl.BlockSpec((rows, d), lambda i: (i, 0)),
            pl.BlockSpec((rows, d), lambda i: (i, 0)),
        ],
        out_shape=[
            jax.ShapeDtypeStruct((g, d), jnp.float32),
            jax.ShapeDtypeStruct((g, d), jnp.int32),
        ],
    )(x2, wqkvT)


def _topk_kernel(crow_ref, ccol_ref, nn_ref, *, goff):
    rb = crow_ref.shape[1]
    n = ccol_ref.shape[2]
    rx = crow_ref[0, :, 0:1]
    ry = crow_ref[0, :, 1:2]
    cx = ccol_ref[0, 0:1, :]
    cy = ccol_ref[0, 1:2, :]
    sqr = rx * rx + ry * ry
    sqc = cx * cx + cy * cy
    rxb = rx.astype(jnp.bfloat16).astype(jnp.float32)
    ryb = ry.astype(jnp.bfloat16).astype(jnp.float32)
    cxb = cx.astype(jnp.bfloat16).astype(jnp.float32)
    cyb = cy.astype(jnp.bfloat16).astype(jnp.float32)
    dot = rxb * cxb + ryb * cyb
    d2 = jnp.maximum((sqr + sqc) - 2.0 * dot, 0.0)
    coliota = lax.broadcasted_iota(jnp.int32, (rb, n), 1)
    bits = lax.bitcast_convert_type(d2, jnp.int32)
    keys = lax.bitcast_convert_type(
        ((bits + jnp.int32(1 << 23)) & jnp.int32(-4096)) | coliota, jnp.float32
    )
    big = jnp.float32(3.0e38)
    for t in range(KSEL):
        kmin = jnp.min(keys, axis=1, keepdims=True)
        idx = lax.bitcast_convert_type(kmin, jnp.int32) & jnp.int32(4095)
        nn_ref[0, :, t : t + 1] = idx + goff
        keys = jnp.where(keys == kmin, big, keys)


def _topk_call(coords, coordsT, bi, off, rows):
    _, n, _ = coords.shape
    rb = 512
    grid = (rows // rb,)
    return pl.pallas_call(
        functools.partial(_topk_kernel, goff=bi * n),
        grid=grid,
        in_specs=[
            pl.BlockSpec((1, rb, 2), lambda j: (bi, off // rb + j, 0)),
            pl.BlockSpec((1, 2, n), lambda j: (bi, 0, 0)),
        ],
        out_specs=pl.BlockSpec((1, rb, KSEL), lambda j: (0, j, 0)),
        out_shape=jax.ShapeDtypeStruct((1, rows, KSEL), jnp.int32),
    )(coords, coordsT)


def _gather_call(kv, idx_flat):
    numi = idx_flat.shape[0]
    dkv = kv.shape[1]
    nw = 32
    b_per_w = numi // nw
    ch = 48
    nch = b_per_w // ch
    mesh = plsc.VectorSubcoreMesh(core_axis_name="c", subcore_axis_name="s")

    @functools.partial(
        pl.kernel,
        out_type=jax.ShapeDtypeStruct((numi, dkv), kv.dtype),
        mesh=mesh,
        scratch_types=[
            pltpu.VMEM((b_per_w,), jnp.int32),
            pltpu.VMEM((ch, dkv), kv.dtype),
            pltpu.VMEM((ch, dkv), kv.dtype),
            pltpu.SemaphoreType.DMA,
            pltpu.SemaphoreType.DMA,
        ],
    )
    def gk(kv_hbm, i_hbm, o_hbm, idx_v, rows0, rows1, sem0, sem1):
        wid = lax.axis_index("s") * 2 + lax.axis_index("c")
        base = wid * b_per_w
        pltpu.sync_copy(i_hbm.at[pl.ds(base, b_per_w)], idx_v)

        def fire(c, buf, sem):
            pltpu.async_copy(kv_hbm.at[idx_v.at[pl.ds(c * ch, ch)]], buf, sem)

        fire(0, rows0, sem0)

        @pl.loop(0, nch, step=2)
        def _(c):
            pltpu.make_async_copy(kv_hbm.at[pl.ds(0, ch)], rows0, sem0).wait()

            @pl.when(c + 1 < nch)
            def _():
                fire(c + 1, rows1, sem1)

            pltpu.sync_copy(rows0, o_hbm.at[pl.ds(base + c * ch, ch)])

            @pl.when(c + 1 < nch)
            def _():
                pltpu.make_async_copy(kv_hbm.at[pl.ds(0, ch)], rows1, sem1).wait()

                @pl.when(c + 2 < nch)
                def _():
                    fire(c + 2, rows0, sem0)

                pltpu.sync_copy(rows1, o_hbm.at[pl.ds(base + (c + 1) * ch, ch)])

    return gk(kv, idx_flat)


def _attn_kernel(q_ref, kv_ref, x_ref, wp_ref, bp_ref, o_ref):
    rd, d = q_ref.shape
    hd = d // HEADS
    q = q_ref[...]
    r1 = lax.broadcasted_iota(jnp.int32, (d, HEADS), 0)
    c1 = lax.broadcasted_iota(jnp.int32, (d, HEADS), 1)
    mhead = (r1 // hd == c1).astype(jnp.float32)
    r2 = lax.broadcasted_iota(jnp.int32, (HEADS, d), 0)
    c2 = lax.broadcasted_iota(jnp.int32, (HEADS, d), 1)
    mspread = (c2 // hd == r2).astype(jnp.float32)

    scale = jnp.float32(1.0 / (hd**0.5))
    logits = []
    for j in range(KSEL):
        kj = lax.bitcast_convert_type(kv_ref[j] << 16, jnp.float32)
        lj = jnp.dot(q * kj, mhead, preferred_element_type=jnp.float32)
        logits.append(lj * scale)
    m = logits[0]
    for j in range(1, KSEL):
        m = jnp.maximum(m, logits[j])
    exps = [jnp.exp(l - m) for l in logits]
    s = exps[0]
    for j in range(1, KSEL):
        s = s + exps[j]
    rs = 1.0 / s
    o = jnp.zeros((rd, d), jnp.float32)
    for j in range(KSEL):
        w = exps[j] * rs
        wexp = jnp.dot(w, mspread, preferred_element_type=jnp.float32)
        vj = lax.bitcast_convert_type(kv_ref[j] & jnp.int32(-65536), jnp.float32)
        o = o + wexp * vj
    g = 0.5 * o * (1.0 + lax.erf(o * jnp.float32(0.7071067811865476)))
    y = jnp.dot(g.astype(jnp.bfloat16), wp_ref[...], preferred_element_type=jnp.float32)
    o_ref[...] = x_ref[...] + y + bp_ref[...]


def _attn_call(q, kvnb3, x2, wpT, bp2, goff, rows):
    _, d = q.shape
    rd = 256
    co = goff // rd
    grid = (rows // rd,)
    return pl.pallas_call(
        _attn_kernel,
        grid=grid,
        in_specs=[
            pl.BlockSpec((rd, d), lambda i: (co + i, 0)),
            pl.BlockSpec((KSEL, rd, d), lambda i: (0, i, 0)),
            pl.BlockSpec((rd, d), lambda i: (co + i, 0)),
            pl.BlockSpec((d, d), lambda i: (0, 0)),
            pl.BlockSpec((1, d), lambda i: (0, 0)),
        ],
        out_specs=pl.BlockSpec((rd, d), lambda i: (i, 0)),
        out_shape=jax.ShapeDtypeStruct((rows, d), jnp.float32),
    )(q, kvnb3, x2, wpT, bp2)


def kernel(x, coords, Wqkv, Wp, bp):
    b, n, d = x.shape
    g = b * n
    x2 = x.reshape(g, d)
    wqkvT = Wqkv.T.astype(jnp.bfloat16)
    wpT = Wp.T.astype(jnp.bfloat16)
    bp2 = bp.reshape(1, d)
    q, kv = _qkv_call(x2, wqkvT)
    coordsT = jnp.transpose(coords, (0, 2, 1))
    rows = n // 4
    outs = []
    for bi in range(b):
        for off in range(0, n, rows):
            nn_c = _topk_call(coords, coordsT, bi, off, rows)
            idx_c = jnp.transpose(nn_c[0], (1, 0)).reshape(KSEL * rows)
            kvnb_c = _gather_call(kv, idx_c)
            kvnb3 = kvnb_c.reshape(KSEL, rows, d)
            outs.append(_attn_call(q, kvnb3, x2, wpT, bp2, bi * n + off, rows))
    out = jnp.concatenate(outs, axis=0)
    return out.reshape(b, n, d)


# speedup vs baseline: 1.0524x; 1.0524x over previous
import functools

import jax
import jax.numpy as jnp
from jax import lax
from jax.experimental import pallas as pl
from jax.experimental.pallas import tpu as pltpu
from jax.experimental.pallas import tpu_sc as plsc

HEADS = 16
KSEL = 9


def _qkv_kernel(x_ref, w_ref, q_ref, kv_ref):
    d = x_ref.shape[1]
    xb = x_ref[...].astype(jnp.bfloat16)
    acc = jnp.dot(xb, w_ref[...], preferred_element_type=jnp.float32)
    q_ref[...] = acc[:, :d]
    kbits = lax.bitcast_convert_type(acc[:, d : 2 * d], jnp.int32)
    vbits = lax.bitcast_convert_type(acc[:, 2 * d :], jnp.int32)
    kv_ref[...] = lax.shift_right_logical(kbits, 16) | (vbits & jnp.int32(-65536))


def _qkv_call(x2, wqkvT):
    g, d = x2.shape
    rows = 512
    grid = (g // rows,)
    return pl.pallas_call(
        _qkv_kernel,
        grid=grid,
        in_specs=[
            pl.BlockSpec((rows, d), lambda i: (i, 0)),
            pl.BlockSpec((d, 3 * d), lambda i: (0, 0)),
        ],
        out_specs=[
            pl.BlockSpec((rows, d), lambda i: (i, 0)),
            pl.BlockSpec((rows, d), lambda i: (i, 0)),
        ],
        out_shape=[
            jax.ShapeDtypeStruct((g, d), jnp.float32),
            jax.ShapeDtypeStruct((g, d), jnp.int32),
        ],
    )(x2, wqkvT)


def _topk_kernel(crow_ref, ccol_ref, nn_ref, *, goff):
    rb = crow_ref.shape[1]
    n = ccol_ref.shape[2]
    rx = crow_ref[0, :, 0:1]
    ry = crow_ref[0, :, 1:2]
    cx = ccol_ref[0, 0:1, :]
    cy = ccol_ref[0, 1:2, :]
    sqr = rx * rx + ry * ry
    sqc = cx * cx + cy * cy
    rxb = rx.astype(jnp.bfloat16).astype(jnp.float32)
    ryb = ry.astype(jnp.bfloat16).astype(jnp.float32)
    cxb = cx.astype(jnp.bfloat16).astype(jnp.float32)
    cyb = cy.astype(jnp.bfloat16).astype(jnp.float32)
    dot = rxb * cxb + ryb * cyb
    d2 = jnp.maximum((sqr + sqc) - 2.0 * dot, 0.0)
    coliota = lax.broadcasted_iota(jnp.int32, (rb, n), 1)
    bits = lax.bitcast_convert_type(d2, jnp.int32)
    keys = lax.bitcast_convert_type(
        ((bits + jnp.int32(1 << 23)) & jnp.int32(-4096)) | coliota, jnp.float32
    )
    big = jnp.float32(3.0e38)
    kmins = []
    for t in range(KSEL):
        kmin = jnp.min(keys, axis=1, keepdims=True)
        kmins.append(kmin)
        keys = jnp.where(keys == kmin, big, keys)
    kmat = jnp.concatenate(kmins, axis=1)
    idx = (lax.bitcast_convert_type(kmat, jnp.int32) & jnp.int32(4095)) + goff
    nn_ref[0, :, :] = jnp.transpose(idx, (1, 0))


def _topk_call(coords, coordsT, bi, off, rows):
    _, n, _ = coords.shape
    rb = 512
    grid = (rows // rb,)
    return pl.pallas_call(
        functools.partial(_topk_kernel, goff=bi * n),
        grid=grid,
        in_specs=[
            pl.BlockSpec((1, rb, 2), lambda j: (bi, off // rb + j, 0)),
            pl.BlockSpec((1, 2, n), lambda j: (bi, 0, 0)),
        ],
        out_specs=pl.BlockSpec((1, KSEL, rb), lambda j: (0, 0, j)),
        out_shape=jax.ShapeDtypeStruct((1, KSEL, rows), jnp.int32),
    )(coords, coordsT)


def _gather_call(kv, idx_flat):
    numi = idx_flat.shape[0]
    dkv = kv.shape[1]
    nw = 32
    b_per_w = numi // nw
    ch = 48
    nch = b_per_w // ch
    mesh = plsc.VectorSubcoreMesh(core_axis_name="c", subcore_axis_name="s")

    @functools.partial(
        pl.kernel,
        out_type=jax.ShapeDtypeStruct((numi, dkv), kv.dtype),
        mesh=mesh,
        scratch_types=[
            pltpu.VMEM((b_per_w,), jnp.int32),
            pltpu.VMEM((ch, dkv), kv.dtype),
            pltpu.VMEM((ch, dkv), kv.dtype),
            pltpu.SemaphoreType.DMA,
            pltpu.SemaphoreType.DMA,
        ],
    )
    def gk(kv_hbm, i_hbm, o_hbm, idx_v, rows0, rows1, sem0, sem1):
        wid = lax.axis_index("s") * 2 + lax.axis_index("c")
        base = wid * b_per_w
        pltpu.sync_copy(i_hbm.at[pl.ds(base, b_per_w)], idx_v)

        def fire(c, buf, sem):
            pltpu.async_copy(kv_hbm.at[idx_v.at[pl.ds(c * ch, ch)]], buf, sem)

        fire(0, rows0, sem0)

        @pl.loop(0, nch, step=2)
        def _(c):
            pltpu.make_async_copy(kv_hbm.at[pl.ds(0, ch)], rows0, sem0).wait()

            @pl.when(c + 1 < nch)
            def _():
                fire(c + 1, rows1, sem1)

            pltpu.sync_copy(rows0, o_hbm.at[pl.ds(base + c * ch, ch)])

            @pl.when(c + 1 < nch)
            def _():
                pltpu.make_async_copy(kv_hbm.at[pl.ds(0, ch)], rows1, sem1).wait()

                @pl.when(c + 2 < nch)
                def _():
                    fire(c + 2, rows0, sem0)

                pltpu.sync_copy(rows1, o_hbm.at[pl.ds(base + (c + 1) * ch, ch)])

    return gk(kv, idx_flat)


def _attn_kernel(q_ref, kv_ref, x_ref, wp_ref, bp_ref, o_ref):
    rd, d = q_ref.shape
    hd = d // HEADS
    q = q_ref[...]
    r1 = lax.broadcasted_iota(jnp.int32, (d, HEADS), 0)
    c1 = lax.broadcasted_iota(jnp.int32, (d, HEADS), 1)
    mhead = (r1 // hd == c1).astype(jnp.float32)
    r2 = lax.broadcasted_iota(jnp.int32, (HEADS, d), 0)
    c2 = lax.broadcasted_iota(jnp.int32, (HEADS, d), 1)
    mspread = (c2 // hd == r2).astype(jnp.float32)

    scale = jnp.float32(1.0 / (hd**0.5))
    logits = []
    for j in range(KSEL):
        kj = lax.bitcast_convert_type(kv_ref[j] << 16, jnp.float32)
        lj = jnp.dot(q * kj, mhead, preferred_element_type=jnp.float32)
        logits.append(lj * scale)
    m = logits[0]
    for j in range(1, KSEL):
        m = jnp.maximum(m, logits[j])
    exps = [jnp.exp(l - m) for l in logits]
    s = exps[0]
    for j in range(1, KSEL):
        s = s + exps[j]
    rs = 1.0 / s
    o = jnp.zeros((rd, d), jnp.float32)
    for j in range(KSEL):
        w = exps[j] * rs
        wexp = jnp.dot(w, mspread, preferred_element_type=jnp.float32)
        vj = lax.bitcast_convert_type(kv_ref[j] & jnp.int32(-65536), jnp.float32)
        o = o + wexp * vj
    g = 0.5 * o * (1.0 + lax.erf(o * jnp.float32(0.7071067811865476)))
    y = jnp.dot(g.astype(jnp.bfloat16), wp_ref[...], preferred_element_type=jnp.float32)
    o_ref[...] = x_ref[...] + y + bp_ref[...]


def _attn_call(q, kvnb3, x2, wpT, bp2, goff, rows):
    _, d = q.shape
    rd = 256
    co = goff // rd
    grid = (rows // rd,)
    return pl.pallas_call(
        _attn_kernel,
        grid=grid,
        in_specs=[
            pl.BlockSpec((rd, d), lambda i: (co + i, 0)),
            pl.BlockSpec((KSEL, rd, d), lambda i: (0, i, 0)),
            pl.BlockSpec((rd, d), lambda i: (co + i, 0)),
            pl.BlockSpec((d, d), lambda i: (0, 0)),
            pl.BlockSpec((1, d), lambda i: (0, 0)),
        ],
        out_specs=pl.BlockSpec((rd, d), lambda i: (i, 0)),
        out_shape=jax.ShapeDtypeStruct((rows, d), jnp.float32),
    )(q, kvnb3, x2, wpT, bp2)


def kernel(x, coords, Wqkv, Wp, bp):
    b, n, d = x.shape
    g = b * n
    x2 = x.reshape(g, d)
    wqkvT = Wqkv.T.astype(jnp.bfloat16)
    wpT = Wp.T.astype(jnp.bfloat16)
    bp2 = bp.reshape(1, d)
    q, kv = _qkv_call(x2, wqkvT)
    coordsT = jnp.transpose(coords, (0, 2, 1))
    rows = n // 2
    outs = []
    for bi in range(b):
        for off in range(0, n, rows):
            nn_c = _topk_call(coords, coordsT, bi, off, rows)
            idx_c = nn_c.reshape(KSEL * rows)
            kvnb_c = _gather_call(kv, idx_c)
            kvnb3 = kvnb_c.reshape(KSEL, rows, d)
            outs.append(_attn_call(q, kvnb3, x2, wpT, bp2, bi * n + off, rows))
    out = jnp.concatenate(outs, axis=0)
    return out.reshape(b, n, d)
